# Initial kernel scaffold; baseline (speedup 1.0000x reference)
import jax
import jax.numpy as jnp
from jax.experimental import pallas as pl


def kernel(x, complete_edge_index, y, W, b):
    raise NotImplementedError("write your pallas kernel here")


# speedup vs baseline: 3.3659x; 3.3659x over previous
import jax
import jax.numpy as jnp
from jax import lax
from jax.experimental import pallas as pl
from jax.experimental.pallas import tpu as pltpu
from jax.experimental.pallas import tpu_sc as plsc

N_NODES = 10000
N_EDGES = 320000
D = 128
NC, NS, L = 2, 16, 16
NW = NC * NS
EW = N_EDGES // NW
B = 80
NBLK = EW // B
NG = B // L


def _ilh_body(x_hbm, src_hbm, dst_hbm, wb_hbm, out_hbm,
              sidx, didx, srow, drow, obuf, wbv, sem):
    wid = lax.axis_index("s") * NC + lax.axis_index("c")
    ebase = wid * EW
    pltpu.sync_copy(wb_hbm, wbv)
    pltpu.sync_copy(src_hbm.at[pl.ds(ebase, EW)], sidx)
    pltpu.sync_copy(dst_hbm.at[pl.ds(ebase, EW)], didx)
    lane = lax.iota(jnp.int32, L)
    w0 = wbv[0, :]
    w1 = wbv[1, :]
    b0 = wbv[2, :]
    b1 = wbv[3, :]

    def block(blk, carry):
        boff = blk * B
        c1 = pltpu.async_copy(x_hbm.at[sidx.at[pl.ds(boff, B)]], srow, sem)
        c2 = pltpu.async_copy(x_hbm.at[didx.at[pl.ds(boff, B)]], drow, sem)
        c1.wait()
        c2.wait()
        for g in range(NG):
            s_vec = jnp.zeros((L,), jnp.float32)
            for e in range(L):
                r = g * L + e
                acc = srow[r, pl.ds(0, L)] * drow[r, pl.ds(0, L)]
                for k in range(1, D // L):
                    acc = acc + srow[r, pl.ds(k * L, L)] * drow[r, pl.ds(k * L, L)]
                s_vec = jnp.where(lane == e, jnp.sum(acc), s_vec)
            i0 = lane * 2 + (2 * L * g)
            plsc.store_scatter(obuf, [i0], s_vec * w0 + b0)
            plsc.store_scatter(obuf, [i0 + 1], s_vec * w1 + b1)
        pltpu.sync_copy(obuf, out_hbm.at[pl.ds(2 * (ebase + boff), 2 * B)])
        return carry

    lax.fori_loop(0, NBLK, block, 0)


def kernel(x, complete_edge_index, y, W, b):
    src = complete_edge_index[0].astype(jnp.int32)
    dst = complete_edge_index[1].astype(jnp.int32)
    wb = jnp.stack([
        jnp.full((L,), W[0, 0], jnp.float32),
        jnp.full((L,), W[1, 0], jnp.float32),
        jnp.full((L,), b[0], jnp.float32),
        jnp.full((L,), b[1], jnp.float32),
    ])
    mesh = plsc.VectorSubcoreMesh(core_axis_name="c", subcore_axis_name="s",
                                  num_cores=NC, num_subcores=NS)
    out = pl.kernel(
        _ilh_body,
        out_type=jax.ShapeDtypeStruct((2 * N_EDGES,), jnp.float32),
        mesh=mesh,
        compiler_params=pltpu.CompilerParams(needs_layout_passes=False),
        scratch_types=[
            pltpu.VMEM((EW,), jnp.int32),
            pltpu.VMEM((EW,), jnp.int32),
            pltpu.VMEM((B, D), jnp.float32),
            pltpu.VMEM((B, D), jnp.float32),
            pltpu.VMEM((2 * B,), jnp.float32),
            pltpu.VMEM((4, L), jnp.float32),
            pltpu.SemaphoreType.DMA,
        ],
    )(x, src, dst, wb)
    y_pred = out.reshape(N_EDGES, 2)
    return (y_pred, y)


# speedup vs baseline: 4.2232x; 1.2547x over previous
import jax
import jax.numpy as jnp
from jax import lax
from jax.experimental import pallas as pl
from jax.experimental.pallas import tpu as pltpu
from jax.experimental.pallas import tpu_sc as plsc

N_NODES = 10000
N_EDGES = 320000
D = 128
NC, NS, L = 2, 16, 16
NW = NC * NS
EW = N_EDGES // NW
B = 80
NBLK = EW // B
NG = B // L


def _ilh_body(x_hbm, src_hbm, dst_hbm, wb_hbm, out_hbm,
              sidx, didx, srow0, srow1, drow0, drow1, ob0, ob1, wbv,
              sg0, sg1, so0, so1):
    srow = (srow0, srow1)
    drow = (drow0, drow1)
    obuf = (ob0, ob1)
    wid = lax.axis_index("s") * NC + lax.axis_index("c")
    ebase = wid * EW
    pltpu.sync_copy(wb_hbm, wbv)
    pltpu.sync_copy(src_hbm.at[pl.ds(ebase, EW)], sidx)
    pltpu.sync_copy(dst_hbm.at[pl.ds(ebase, EW)], didx)
    lane = lax.iota(jnp.int32, L)
    w0 = wbv[0, :]
    w1 = wbv[1, :]
    b0 = wbv[2, :]
    b1 = wbv[3, :]
    sg = (sg0, sg1)
    so = (so0, so1)

    def issue(blk, slot):
        boff = blk * B
        pltpu.async_copy(x_hbm.at[sidx.at[pl.ds(boff, B)]], srow[slot], sg[slot])
        pltpu.async_copy(x_hbm.at[didx.at[pl.ds(boff, B)]], drow[slot], sg[slot])

    def wait_gather(blk, slot):
        boff = blk * B
        pltpu.make_async_copy(
            x_hbm.at[sidx.at[pl.ds(boff, B)]], srow[slot], sg[slot]).wait()
        pltpu.make_async_copy(
            x_hbm.at[didx.at[pl.ds(boff, B)]], drow[slot], sg[slot]).wait()

    def out_ref(blk):
        return out_hbm.at[pl.ds(2 * (ebase + blk * B), 2 * B)]

    def out_wait(blk, slot):
        pltpu.make_async_copy(obuf[slot], out_ref(blk), so[slot]).wait()

    def compute(blk, slot):
        sr = srow[slot]
        dr = drow[slot]
        ob = obuf[slot]

        def group(g, c):
            s_vec = jnp.zeros((L,), jnp.float32)
            base_r = g * L
            for e in range(L):
                r = base_r + e
                p = [sr[r, pl.ds(k * L, L)] * dr[r, pl.ds(k * L, L)]
                     for k in range(D // L)]
                acc = ((p[0] + p[1]) + (p[2] + p[3])) + ((p[4] + p[5]) + (p[6] + p[7]))
                s_vec = jnp.where(lane == e, jnp.sum(acc), s_vec)
            i0 = lane * 2 + 2 * L * g
            plsc.store_scatter(ob, [i0], s_vec * w0 + b0)
            plsc.store_scatter(ob, [i0 + 1], s_vec * w1 + b1)
            return c

        lax.fori_loop(0, NG, group, 0)
        pltpu.async_copy(ob, out_ref(blk), so[slot])

    issue(0, 0)
    issue(1, 1)

    def pair(i, c):
        blk0 = 2 * i
        blk1 = 2 * i + 1
        wait_gather(blk0, 0)

        @pl.when(i > 0)
        def _():
            out_wait(blk0 - 2, 0)

        compute(blk0, 0)
        issue(blk0 + 2, 0)
        wait_gather(blk1, 1)

        @pl.when(i > 0)
        def _():
            out_wait(blk1 - 2, 1)

        compute(blk1, 1)

        @pl.when(blk1 + 2 < NBLK)
        def _():
            issue(blk1 + 2, 1)

        return c

    lax.fori_loop(0, (NBLK - 1) // 2, pair, 0)

    wait_gather(NBLK - 1, 0)
    out_wait(NBLK - 3, 0)
    compute(NBLK - 1, 0)
    out_wait(NBLK - 2, 1)
    out_wait(NBLK - 1, 0)


def kernel(x, complete_edge_index, y, W, b):
    src = complete_edge_index[0].astype(jnp.int32)
    dst = complete_edge_index[1].astype(jnp.int32)
    wb = jnp.stack([
        jnp.full((L,), W[0, 0], jnp.float32),
        jnp.full((L,), W[1, 0], jnp.float32),
        jnp.full((L,), b[0], jnp.float32),
        jnp.full((L,), b[1], jnp.float32),
    ])
    mesh = plsc.VectorSubcoreMesh(core_axis_name="c", subcore_axis_name="s",
                                  num_cores=NC, num_subcores=NS)
    out = pl.kernel(
        _ilh_body,
        out_type=jax.ShapeDtypeStruct((2 * N_EDGES,), jnp.float32),
        mesh=mesh,
        compiler_params=pltpu.CompilerParams(needs_layout_passes=False),
        scratch_types=[
            pltpu.VMEM((EW,), jnp.int32),
            pltpu.VMEM((EW,), jnp.int32),
            pltpu.VMEM((B, D), jnp.float32),
            pltpu.VMEM((B, D), jnp.float32),
            pltpu.VMEM((B, D), jnp.float32),
            pltpu.VMEM((B, D), jnp.float32),
            pltpu.VMEM((2 * B,), jnp.float32),
            pltpu.VMEM((2 * B,), jnp.float32),
            pltpu.VMEM((4, L), jnp.float32),
            pltpu.SemaphoreType.DMA,
            pltpu.SemaphoreType.DMA,
            pltpu.SemaphoreType.DMA,
            pltpu.SemaphoreType.DMA,
        ],
    )(x, src, dst, wb)
    y_pred = out.reshape(N_EDGES, 2)
    return (y_pred, y)


# speedup vs baseline: 6.6824x; 1.5823x over previous
import jax
import jax.numpy as jnp
from jax import lax
from jax.experimental import pallas as pl
from jax.experimental.pallas import tpu as pltpu
from jax.experimental.pallas import tpu_sc as plsc

N_NODES = 10000
N_EDGES = 320000
D = 128
NC, NS, L = 2, 16, 16
NW = NC * NS
EW = N_EDGES // NW
B = 80
NBLK = EW // B
NG = B // L


def _ilh_body(x_hbm, src_hbm, dst_hbm, wb_hbm, out0_hbm, out1_hbm,
              sidx, didx, srow0, srow1, drow0, drow1, o0, o1, wbv,
              sg0, sg1):
    srow = (srow0, srow1)
    drow = (drow0, drow1)
    sg = (sg0, sg1)
    wid = lax.axis_index("s") * NC + lax.axis_index("c")
    ebase = wid * EW
    pltpu.sync_copy(wb_hbm, wbv)
    pltpu.sync_copy(src_hbm.at[pl.ds(ebase, EW)], sidx)
    pltpu.sync_copy(dst_hbm.at[pl.ds(ebase, EW)], didx)
    lane = lax.iota(jnp.int32, L)
    w0 = wbv[0, :]
    w1 = wbv[1, :]
    b0 = wbv[2, :]
    b1 = wbv[3, :]

    def issue(blk, slot):
        boff = blk * B
        pltpu.async_copy(x_hbm.at[sidx.at[pl.ds(boff, B)]], srow[slot], sg[slot])
        pltpu.async_copy(x_hbm.at[didx.at[pl.ds(boff, B)]], drow[slot], sg[slot])

    def wait_gather(blk, slot):
        boff = blk * B
        pltpu.make_async_copy(
            x_hbm.at[sidx.at[pl.ds(boff, B)]], srow[slot], sg[slot]).wait()
        pltpu.make_async_copy(
            x_hbm.at[didx.at[pl.ds(boff, B)]], drow[slot], sg[slot]).wait()

    def compute(blk, slot):
        sr = srow[slot]
        dr = drow[slot]
        boff = blk * B

        def group(g, c):
            s_vec = jnp.zeros((L,), jnp.float32)
            base_r = g * L
            for e in range(L):
                r = base_r + e
                p = [sr[r, pl.ds(k * L, L)] * dr[r, pl.ds(k * L, L)]
                     for k in range(D // L)]
                acc = ((p[0] + p[1]) + (p[2] + p[3])) + ((p[4] + p[5]) + (p[6] + p[7]))
                s_vec = jnp.where(lane == e, jnp.sum(acc), s_vec)
            o0[pl.ds(boff + g * L, L)] = s_vec * w0 + b0
            o1[pl.ds(boff + g * L, L)] = s_vec * w1 + b1
            return c

        lax.fori_loop(0, NG, group, 0)

    issue(0, 0)
    issue(1, 1)

    def pair(i, c):
        blk0 = 2 * i
        blk1 = 2 * i + 1
        wait_gather(blk0, 0)
        compute(blk0, 0)
        issue(blk0 + 2, 0)
        wait_gather(blk1, 1)
        compute(blk1, 1)

        @pl.when(blk1 + 2 < NBLK)
        def _():
            issue(blk1 + 2, 1)

        return c

    lax.fori_loop(0, (NBLK - 1) // 2, pair, 0)

    wait_gather(NBLK - 1, 0)
    compute(NBLK - 1, 0)
    pltpu.sync_copy(o0, out0_hbm.at[pl.ds(ebase, EW)])
    pltpu.sync_copy(o1, out1_hbm.at[pl.ds(ebase, EW)])


def kernel(x, complete_edge_index, y, W, b):
    ei = complete_edge_index.astype(jnp.int32)
    src_idx = ei[0]
    dst_idx = ei[1]
    wb = jnp.stack([
        jnp.full((L,), W[0, 0], jnp.float32),
        jnp.full((L,), W[1, 0], jnp.float32),
        jnp.full((L,), b[0], jnp.float32),
        jnp.full((L,), b[1], jnp.float32),
    ])
    mesh = plsc.VectorSubcoreMesh(core_axis_name="c", subcore_axis_name="s",
                                  num_cores=NC, num_subcores=NS)
    out0, out1 = pl.kernel(
        _ilh_body,
        out_type=(jax.ShapeDtypeStruct((N_EDGES,), jnp.float32),
                  jax.ShapeDtypeStruct((N_EDGES,), jnp.float32)),
        mesh=mesh,
        compiler_params=pltpu.CompilerParams(needs_layout_passes=False),
        scratch_types=[
            pltpu.VMEM((EW,), jnp.int32),
            pltpu.VMEM((EW,), jnp.int32),
            pltpu.VMEM((B, D), jnp.float32),
            pltpu.VMEM((B, D), jnp.float32),
            pltpu.VMEM((B, D), jnp.float32),
            pltpu.VMEM((B, D), jnp.float32),
            pltpu.VMEM((EW,), jnp.float32),
            pltpu.VMEM((EW,), jnp.float32),
            pltpu.VMEM((4, L), jnp.float32),
            pltpu.SemaphoreType.DMA,
            pltpu.SemaphoreType.DMA,
        ],
    )(x, src_idx, dst_idx, wb)
    y_pred = jnp.stack([out0, out1], axis=1)
    return (y_pred, y)


# speedup vs baseline: 8.7269x; 1.3059x over previous
import jax
import jax.numpy as jnp
from jax import lax
from jax.experimental import pallas as pl
from jax.experimental.pallas import tpu as pltpu
from jax.experimental.pallas import tpu_sc as plsc

N_NODES = 10000
N_EDGES = 320000
D = 128
DW = D // 2
NC, NS, L = 2, 16, 16
NW = NC * NS
EW = N_EDGES // NW
B = 80
NBLK = EW // B
NG = B // L


def _ilh_body(xp_hbm, src_hbm, dst_hbm, wb_hbm, out0_hbm, out1_hbm,
              sidx, didx, srow0, srow1, drow0, drow1, o0, o1, wbv,
              sg0, sg1):
    srow = (srow0, srow1)
    drow = (drow0, drow1)
    sg = (sg0, sg1)
    wid = lax.axis_index("s") * NC + lax.axis_index("c")
    ebase = wid * EW
    pltpu.sync_copy(wb_hbm, wbv)
    pltpu.sync_copy(src_hbm.at[pl.ds(ebase, EW)], sidx)
    pltpu.sync_copy(dst_hbm.at[pl.ds(ebase, EW)], didx)
    lane = lax.iota(jnp.int32, L)
    m15 = lane == (L - 1)
    w0 = wbv[0, :]
    w1 = wbv[1, :]
    b0 = wbv[2, :]
    b1 = wbv[3, :]

    def issue(blk, slot):
        boff = blk * B
        pltpu.async_copy(xp_hbm.at[sidx.at[pl.ds(boff, B)]], srow[slot], sg[slot])
        pltpu.async_copy(xp_hbm.at[didx.at[pl.ds(boff, B)]], drow[slot], sg[slot])

    def wait_gather(blk, slot):
        boff = blk * B
        pltpu.make_async_copy(
            xp_hbm.at[sidx.at[pl.ds(boff, B)]], srow[slot], sg[slot]).wait()
        pltpu.make_async_copy(
            xp_hbm.at[didx.at[pl.ds(boff, B)]], drow[slot], sg[slot]).wait()

    def compute(blk, slot):
        sr = srow[slot]
        dr = drow[slot]
        boff = blk * B

        def group(g, c):
            base_r = g * L
            for e in range(L):
                r = base_r + e
                acc_a = sr[r, pl.ds(0, L)] * dr[r, pl.ds(0, L)]
                acc_b = sr[r, pl.ds(L, L)] * dr[r, pl.ds(L, L)]
                for k in range(2, D // L, 2):
                    acc_a = acc_a + sr[r, pl.ds(k * L, L)] * dr[r, pl.ds(k * L, L)]
                    acc_b = acc_b + sr[r, pl.ds((k + 1) * L, L)] * dr[r, pl.ds((k + 1) * L, L)]
                cum = plsc.cumsum(acc_a + acc_b)
                idx = boff + r
                plsc.store_compressed(o0.at[pl.ds(idx, L)], cum * w0 + b0, mask=m15)
                plsc.store_compressed(o1.at[pl.ds(idx, L)], cum * w1 + b1, mask=m15)
            return c

        lax.fori_loop(0, NG, group, 0)

    issue(0, 0)
    issue(1, 1)

    def pair(i, c):
        blk0 = 2 * i
        blk1 = 2 * i + 1
        wait_gather(blk0, 0)
        compute(blk0, 0)
        issue(blk0 + 2, 0)
        wait_gather(blk1, 1)
        compute(blk1, 1)

        @pl.when(blk1 + 2 < NBLK)
        def _():
            issue(blk1 + 2, 1)

        return c

    lax.fori_loop(0, (NBLK - 1) // 2, pair, 0)

    wait_gather(NBLK - 1, 0)
    compute(NBLK - 1, 0)
    pltpu.sync_copy(o0.at[pl.ds(0, EW)], out0_hbm.at[pl.ds(ebase, EW)])
    pltpu.sync_copy(o1.at[pl.ds(0, EW)], out1_hbm.at[pl.ds(ebase, EW)])


def kernel(x, complete_edge_index, y, W, b):
    ei = complete_edge_index.astype(jnp.int32)
    src_idx = ei[0]
    dst_idx = ei[1]
    xp = x
    wb = jnp.stack([
        jnp.full((L,), W[0, 0], jnp.float32),
        jnp.full((L,), W[1, 0], jnp.float32),
        jnp.full((L,), b[0], jnp.float32),
        jnp.full((L,), b[1], jnp.float32),
    ])
    mesh = plsc.VectorSubcoreMesh(core_axis_name="c", subcore_axis_name="s",
                                  num_cores=NC, num_subcores=NS)
    out0, out1 = pl.kernel(
        _ilh_body,
        out_type=(jax.ShapeDtypeStruct((N_EDGES,), jnp.float32),
                  jax.ShapeDtypeStruct((N_EDGES,), jnp.float32)),
        mesh=mesh,
        compiler_params=pltpu.CompilerParams(needs_layout_passes=False),
        scratch_types=[
            pltpu.VMEM((EW,), jnp.int32),
            pltpu.VMEM((EW,), jnp.int32),
            pltpu.VMEM((B, D), jnp.float32),
            pltpu.VMEM((B, D), jnp.float32),
            pltpu.VMEM((B, D), jnp.float32),
            pltpu.VMEM((B, D), jnp.float32),
            pltpu.VMEM((EW + L,), jnp.float32),
            pltpu.VMEM((EW + L,), jnp.float32),
            pltpu.VMEM((4, L), jnp.float32),
            pltpu.SemaphoreType.DMA,
            pltpu.SemaphoreType.DMA,
        ],
    )(xp, src_idx, dst_idx, wb)
    y_pred = jnp.stack([out0, out1], axis=1)
    return (y_pred, y)


# speedup vs baseline: 15.7228x; 1.8017x over previous
import jax
import jax.numpy as jnp
from jax import lax
from jax.experimental import pallas as pl
from jax.experimental.pallas import tpu as pltpu
from jax.experimental.pallas import tpu_sc as plsc

N_NODES = 10000
N_EDGES = 320000
D = 128
NC, NS, L = 2, 16, 16
NW = NC * NS
EW = N_EDGES // NW
B = 80
NBLK = EW // B
NG = B // L


def _ilh_body(xp_hbm, ei_hbm, wb_hbm, out0_hbm, out1_hbm,
              sidx, didx, srow0, srow1, srow2, srow3,
              drow0, drow1, drow2, drow3, o0, o1, wbv,
              sg0, sg1, sg2, sg3):
    srow = (srow0, srow1, srow2, srow3)
    drow = (drow0, drow1, drow2, drow3)
    sg = (sg0, sg1, sg2, sg3)
    wid = lax.axis_index("s") * NC + lax.axis_index("c")
    ebase = wid * EW
    pltpu.sync_copy(wb_hbm, wbv)
    pltpu.sync_copy(ei_hbm.at[pl.ds(ebase, EW)], sidx)
    pltpu.sync_copy(ei_hbm.at[pl.ds(N_EDGES + ebase, EW)], didx)
    lane = lax.iota(jnp.int32, L)
    rots = [(lane + sh) & (L - 1) for sh in (8, 4, 2, 1)]
    w0 = wbv[0, :]
    w1 = wbv[1, :]
    b0 = wbv[2, :]
    b1 = wbv[3, :]

    def issue(blk, slot):
        boff = blk * B
        pltpu.async_copy(xp_hbm.at[sidx.at[pl.ds(boff, B)]], srow[slot], sg[slot])
        pltpu.async_copy(xp_hbm.at[didx.at[pl.ds(boff, B)]], drow[slot], sg[slot])

    def wait_gather(blk, slot):
        boff = blk * B
        pltpu.make_async_copy(
            xp_hbm.at[sidx.at[pl.ds(boff, B)]], srow[slot], sg[slot]).wait()
        pltpu.make_async_copy(
            xp_hbm.at[didx.at[pl.ds(boff, B)]], drow[slot], sg[slot]).wait()

    def compute(blk, slot):
        sr = srow[slot]
        dr = drow[slot]
        boff = blk * B

        def group(g, c):
            base_r = g * L

            def edge2(e2, s_vec):
                for half in range(2):
                    e = e2 * 2 + half
                    r = base_r + e
                    acc_a = sr[r, pl.ds(0, L)] * dr[r, pl.ds(0, L)]
                    acc_b = sr[r, pl.ds(L, L)] * dr[r, pl.ds(L, L)]
                    for k in range(2, D // L, 2):
                        acc_a = acc_a + sr[r, pl.ds(k * L, L)] * dr[r, pl.ds(k * L, L)]
                        acc_b = acc_b + sr[r, pl.ds((k + 1) * L, L)] * dr[r, pl.ds((k + 1) * L, L)]
                    cum = acc_a + acc_b
                    for rv in rots:
                        cum = cum + lax.gather(
                            cum, rv[:, None],
                            lax.GatherDimensionNumbers(
                                offset_dims=(), collapsed_slice_dims=(0,),
                                start_index_map=(0,)),
                            (1,), mode=lax.GatherScatterMode.PROMISE_IN_BOUNDS)
                    s_vec = jnp.where(lane == e, cum, s_vec)
                return s_vec

            s_vec = lax.fori_loop(0, L // 2, edge2,
                                  jnp.zeros((L,), jnp.float32), unroll=False)
            o0[pl.ds(boff + base_r, L)] = s_vec * w0 + b0
            o1[pl.ds(boff + base_r, L)] = s_vec * w1 + b1
            return c

        lax.fori_loop(0, NG, group, 0)

    for j in range(4):
        issue(j, j)

    def quad(i, c):
        for j in range(4):
            blk = 4 * i + j
            wait_gather(blk, j)
            compute(blk, j)

            @pl.when(blk + 4 < NBLK)
            def _():
                issue(blk + 4, j)

        return c

    lax.fori_loop(0, (NBLK - 1) // 4, quad, 0)

    wait_gather(NBLK - 1, 0)
    compute(NBLK - 1, 0)
    pltpu.sync_copy(o0.at[pl.ds(0, EW)], out0_hbm.at[pl.ds(ebase, EW)])
    pltpu.sync_copy(o1.at[pl.ds(0, EW)], out1_hbm.at[pl.ds(ebase, EW)])


def kernel(x, complete_edge_index, y, W, b):
    ei_flat = complete_edge_index.astype(jnp.int32).reshape(2 * N_EDGES)
    wb = jnp.stack([
        jnp.full((L,), W[0, 0], jnp.float32),
        jnp.full((L,), W[1, 0], jnp.float32),
        jnp.full((L,), b[0], jnp.float32),
        jnp.full((L,), b[1], jnp.float32),
    ])
    mesh = plsc.VectorSubcoreMesh(core_axis_name="c", subcore_axis_name="s",
                                  num_cores=NC, num_subcores=NS)
    out0, out1 = pl.kernel(
        _ilh_body,
        out_type=(jax.ShapeDtypeStruct((N_EDGES,), jnp.float32),
                  jax.ShapeDtypeStruct((N_EDGES,), jnp.float32)),
        mesh=mesh,
        compiler_params=pltpu.CompilerParams(needs_layout_passes=False),
        scratch_types=[
            pltpu.VMEM((EW,), jnp.int32),
            pltpu.VMEM((EW,), jnp.int32),
            pltpu.VMEM((B, D), jnp.float32),
            pltpu.VMEM((B, D), jnp.float32),
            pltpu.VMEM((B, D), jnp.float32),
            pltpu.VMEM((B, D), jnp.float32),
            pltpu.VMEM((B, D), jnp.float32),
            pltpu.VMEM((B, D), jnp.float32),
            pltpu.VMEM((B, D), jnp.float32),
            pltpu.VMEM((B, D), jnp.float32),
            pltpu.VMEM((EW + L,), jnp.float32),
            pltpu.VMEM((EW + L,), jnp.float32),
            pltpu.VMEM((4, L), jnp.float32),
            pltpu.SemaphoreType.DMA,
            pltpu.SemaphoreType.DMA,
            pltpu.SemaphoreType.DMA,
            pltpu.SemaphoreType.DMA,
        ],
    )(x, ei_flat, wb)
    y_pred = jnp.stack([out0, out1], axis=1)
    return (y_pred, y)
